# Initial kernel scaffold; baseline (speedup 1.0000x reference)
import jax
import jax.numpy as jnp
from jax.experimental import pallas as pl


def kernel(h, edge_index, e, snorm_n, snorm_e, emb, W1, b1, Wp, bp, Wm0, bm0, Wm1, bm1, Wm2, bm2):
    raise NotImplementedError("write your pallas kernel here")


# speedup vs baseline: 4.1159x; 4.1159x over previous
import functools

import jax
import jax.numpy as jnp
from jax import lax
from jax.experimental import pallas as pl
from jax.experimental.pallas import tpu as pltpu
from jax.experimental.pallas import tpu_sc as plsc

N = 10000
E = 320000
HID = 128
FH = HID // 2
NC_OUT = 6

NTILES = 32
NP = 10240
EP = 16 * 160 * 128
ROWS_PER_TILE = NP // 16

_f32 = jnp.float32


def _mesh():
    return plsc.VectorSubcoreMesh(core_axis_name="c", subcore_axis_name="s")


def _zero_vmem_2d(buf, nrows, width):
    @pl.loop(0, nrows)
    def _(i):
        for k in range(width // 16):
            buf[i, pl.ds(k * 16, 16)] = jnp.zeros((16,), _f32)


def _sc_embed_deg_body(emb_hbm, hp_hbm, dst_hbm, g_out, degp_out,
                       hidx_v, dstv, buf, ones_v, dbuf, acc1, sem):
    c = lax.axis_index("c")
    s = lax.axis_index("s")
    t = c * 16 + s

    @pl.loop(0, ROWS_PER_TILE // 16)
    def _(i):
        dbuf[pl.ds(i * 16, 16)] = jnp.zeros((16,), _f32)
    pltpu.sync_copy(dbuf, acc1.at[pl.ds(s * ROWS_PER_TILE, ROWS_PER_TILE)])
    for k in range(8):
        ones_v[pl.ds(k * 16, 16)] = jnp.ones((16,), _f32)
    plsc.subcore_barrier()

    pltpu.sync_copy(hp_hbm.at[t], hidx_v)
    @pl.loop(0, 4)
    def _(j):
        pltpu.async_copy(emb_hbm.at[hidx_v.at[j]], buf.at[pl.ds(0, 80)],
                         sem).wait()
        pltpu.sync_copy(buf.at[pl.ds(0, 80)],
                        g_out.at[pl.ds(t * 320 + j * 80, 80)])

    pltpu.sync_copy(dst_hbm.at[t], dstv)
    @pl.loop(0, 80)
    def _(j):
        pltpu.sync_copy(ones_v, acc1.at[dstv.at[j]], add=True)
    plsc.subcore_barrier()

    pltpu.sync_copy(acc1.at[pl.ds(s * ROWS_PER_TILE, ROWS_PER_TILE)], dbuf)
    pltpu.sync_copy(dbuf, degp_out.at[c, pl.ds(s * ROWS_PER_TILE,
                                               ROWS_PER_TILE)])


@functools.cache
def _sc_embed_deg():
    return pl.kernel(
        _sc_embed_deg_body,
        out_type=[jax.ShapeDtypeStruct((NP, HID), _f32),
                  jax.ShapeDtypeStruct((2, NP), _f32)],
        mesh=_mesh(),
        scratch_types=[
            pltpu.VMEM((4, 80), jnp.int32),
            pltpu.VMEM((80, 128), jnp.int32),
            pltpu.VMEM((128, HID), _f32),
            pltpu.VMEM((128,), _f32),
            pltpu.VMEM((ROWS_PER_TILE,), _f32),
            pltpu.VMEM_SHARED((NP,), _f32),
            pltpu.SemaphoreType.DMA,
        ],
    )


def _sc_prop_body(x_hbm, src_hbm, dst_hbm, out_hbm,
                  srcv, dstv, buf0, buf1, buf2, buf3, acc,
                  sem0, sem1, sem2, sem3):
    c = lax.axis_index("c")
    s = lax.axis_index("s")
    rowbase = s * ROWS_PER_TILE
    bufs = (buf0, buf1, buf2, buf3)
    sems = (sem0, sem1, sem2, sem3)

    _zero_vmem_2d(buf0, 128, FH)
    for j in range(5):
        pltpu.sync_copy(buf0, acc.at[pl.ds(rowbase + j * 128, 128)])
    plsc.subcore_barrier()

    pltpu.sync_copy(src_hbm.at[s], srcv)
    pltpu.sync_copy(dst_hbm.at[s], dstv)

    xview = x_hbm.at[pl.ds(c * NP, NP)]

    @pl.loop(0, 160, step=4)
    def _(jj):
        descs = [pltpu.async_copy(xview.at[srcv.at[jj + b]], bufs[b],
                                  sems[b]) for b in range(4)]
        for b in range(4):
            descs[b].wait()
            pltpu.sync_copy(bufs[b], acc.at[dstv.at[jj + b]], add=True)

    plsc.subcore_barrier()

    for j in range(5):
        pltpu.sync_copy(acc.at[pl.ds(rowbase + j * 128, 128)], buf0)
        pltpu.sync_copy(buf0, out_hbm.at[c, pl.ds(rowbase + j * 128, 128)])


@functools.cache
def _sc_prop():
    return pl.kernel(
        _sc_prop_body,
        out_type=jax.ShapeDtypeStruct((2, NP, FH), _f32),
        mesh=_mesh(),
        scratch_types=[
            pltpu.VMEM((160, 128), jnp.int32),
            pltpu.VMEM((160, 128), jnp.int32),
            pltpu.VMEM((128, FH), _f32),
            pltpu.VMEM((128, FH), _f32),
            pltpu.VMEM((128, FH), _f32),
            pltpu.VMEM((128, FH), _f32),
            pltpu.VMEM_SHARED((NP, FH), _f32),
            pltpu.SemaphoreType.DMA,
            pltpu.SemaphoreType.DMA,
            pltpu.SemaphoreType.DMA,
            pltpu.SemaphoreType.DMA,
        ],
        compiler_params=pltpu.CompilerParams(use_tc_tiling_on_sc=False),
    )


_BLK = 1024
_GRID = NP // _BLK


def _norm_from_deg(d0, d1):
    deg = jnp.maximum(d0 + d1, 1.0)
    return lax.rsqrt(deg)


def _tc1_body(g_ref, d0_ref, d1_ref, w_ref, b_ref, o_ref):
    norm = _norm_from_deg(d0_ref[...], d1_ref[...])
    x = jnp.dot(g_ref[...], w_ref[0], preferred_element_type=_f32)
    o_ref[...] = (x + b_ref[0]) * norm


def _tc2_body(p_ref, d0_ref, d1_ref, o_ref):
    norm = _norm_from_deg(d0_ref[...], d1_ref[...])
    o_ref[...] = p_ref[0] * (norm * norm)


def _tc3_body(q_ref, d0_ref, d1_ref, wp_ref, bp_ref,
              w0_ref, b0_ref, w1_ref, b1_ref, w2_ref, b2_ref, o_ref):
    norm = _norm_from_deg(d0_ref[...], d1_ref[...])
    x = jnp.concatenate([q_ref[0], q_ref[1]], axis=1) * norm
    z = jnp.dot(x, wp_ref[...], preferred_element_type=_f32) + bp_ref[...]
    a = jnp.maximum(jnp.dot(z, w0_ref[...], preferred_element_type=_f32)
                    + b0_ref[...], 0.0)
    b = jnp.maximum(jnp.dot(a, w1_ref[...], preferred_element_type=_f32)
                    + b1_ref[...], 0.0)
    o_ref[...] = jnp.dot(b, w2_ref[...], preferred_element_type=_f32) \
        + b2_ref[...]


def _row_spec(width):
    return pl.BlockSpec((_BLK, width), lambda i: (i, 0))


def _full_spec(shape):
    return pl.BlockSpec(shape, lambda i: tuple(0 for _ in shape))


_tc1 = pl.pallas_call(
    _tc1_body,
    grid=(2, _GRID),
    in_specs=[
        pl.BlockSpec((_BLK, HID), lambda c, g: (g, 0)),
        pl.BlockSpec((_BLK, 1), lambda c, g: (g, 0)),
        pl.BlockSpec((_BLK, 1), lambda c, g: (g, 0)),
        pl.BlockSpec((1, HID, FH), lambda c, g: (c, 0, 0)),
        pl.BlockSpec((1, 1, FH), lambda c, g: (c, 0, 0)),
    ],
    out_specs=pl.BlockSpec((_BLK, FH), lambda c, g: (c * _GRID + g, 0)),
    out_shape=jax.ShapeDtypeStruct((2 * NP, FH), _f32),
)

_tc2 = pl.pallas_call(
    _tc2_body,
    grid=(2, _GRID),
    in_specs=[
        pl.BlockSpec((1, _BLK, FH), lambda c, g: (c, g, 0)),
        pl.BlockSpec((_BLK, 1), lambda c, g: (g, 0)),
        pl.BlockSpec((_BLK, 1), lambda c, g: (g, 0)),
    ],
    out_specs=pl.BlockSpec((_BLK, FH), lambda c, g: (c * _GRID + g, 0)),
    out_shape=jax.ShapeDtypeStruct((2 * NP, FH), _f32),
)

_tc3 = pl.pallas_call(
    _tc3_body,
    grid=(_GRID,),
    in_specs=[pl.BlockSpec((2, _BLK, FH), lambda g: (0, g, 0)),
              pl.BlockSpec((_BLK, 1), lambda g: (g, 0)),
              pl.BlockSpec((_BLK, 1), lambda g: (g, 0))]
    + [_full_spec((HID, HID)), _full_spec((1, HID))] * 4,
    out_specs=_row_spec(HID),
    out_shape=jax.ShapeDtypeStruct((NP, HID), _f32),
)


def _pad_w(w):
    r, ccols = w.shape
    return jnp.pad(w, ((0, HID - r), (0, HID - ccols)))


def _pad_b(b):
    return jnp.pad(b, (0, HID - b.shape[0])).reshape(1, HID)


def kernel(h, edge_index, e, snorm_n, snorm_e, emb, W1, b1, Wp, bp,
           Wm0, bm0, Wm1, bm1, Wm2, bm2):
    h_p = jnp.pad(h.astype(jnp.int32), (0, NP - N)).reshape(NTILES, 4, 80)
    src = edge_index[0].astype(jnp.int32)
    dst = edge_index[1].astype(jnp.int32)
    src_p = jnp.pad(src, (0, EP - E))
    dst_p = jnp.pad(dst, (0, EP - E), constant_values=NP - 1)
    src16 = src_p.reshape(16, 160, 128)
    dst16 = dst_p.reshape(16, 160, 128)
    dst32 = dst_p.reshape(NTILES, 80, 128)

    g, degp = _sc_embed_deg()(emb, h_p, dst32)
    d0 = degp[0].reshape(NP, 1)
    d1 = degp[1].reshape(NP, 1)

    W1s = jnp.stack([W1[:, :FH], W1[:, FH:]])
    b1s = jnp.stack([b1[:FH], b1[FH:]]).reshape(2, 1, FH)
    xs = _tc1(g, d0, d1, W1s, b1s)

    p = _sc_prop()(xs, src16, dst16)
    xs2 = _tc2(p, d0, d1)
    q = _sc_prop()(xs2, src16, dst16)

    outp = _tc3(q, d0, d1, Wp, bp.reshape(1, HID),
                _pad_w(Wm0), _pad_b(bm0), _pad_w(Wm1), _pad_b(bm1),
                _pad_w(Wm2), _pad_b(bm2))
    return outp[:N, :NC_OUT]


# speedup vs baseline: 4.3245x; 1.0507x over previous
import functools

import jax
import jax.numpy as jnp
from jax import lax
from jax.experimental import pallas as pl
from jax.experimental.pallas import tpu as pltpu
from jax.experimental.pallas import tpu_sc as plsc

N = 10000
E = 320000
HID = 128
FH = HID // 2
NC_OUT = 6

NTILES = 32
NP = 10240
EP = 16 * 160 * 128
ROWS_PER_TILE = NP // 16

_f32 = jnp.float32


def _mesh():
    return plsc.VectorSubcoreMesh(core_axis_name="c", subcore_axis_name="s")


def _zero_vmem_2d(buf, nrows, width):
    @pl.loop(0, nrows)
    def _(i):
        for k in range(width // 16):
            buf[i, pl.ds(k * 16, 16)] = jnp.zeros((16,), _f32)


def _sc_embed_deg_body(emb_hbm, hp_hbm, dst_hbm, g_out, degp_out,
                       hidx_v, dstv, buf, ones_v, dbuf, acc1, sem):
    c = lax.axis_index("c")
    s = lax.axis_index("s")
    t = c * 16 + s

    @pl.loop(0, ROWS_PER_TILE // 16)
    def _(i):
        dbuf[pl.ds(i * 16, 16)] = jnp.zeros((16,), _f32)
    pltpu.sync_copy(dbuf, acc1.at[pl.ds(s * ROWS_PER_TILE, ROWS_PER_TILE)])
    for k in range(8):
        ones_v[pl.ds(k * 16, 16)] = jnp.ones((16,), _f32)
    plsc.subcore_barrier()

    pltpu.sync_copy(hp_hbm.at[t], hidx_v)
    @pl.loop(0, 4)
    def _(j):
        pltpu.async_copy(emb_hbm.at[hidx_v.at[j]], buf.at[pl.ds(0, 80)],
                         sem).wait()
        pltpu.sync_copy(buf.at[pl.ds(0, 80)],
                        g_out.at[pl.ds(t * 320 + j * 80, 80)])

    pltpu.sync_copy(dst_hbm.at[t], dstv)
    @pl.loop(0, 80)
    def _(j):
        pltpu.sync_copy(ones_v, acc1.at[dstv.at[j]], add=True)
    plsc.subcore_barrier()

    pltpu.sync_copy(acc1.at[pl.ds(s * ROWS_PER_TILE, ROWS_PER_TILE)], dbuf)
    pltpu.sync_copy(dbuf, degp_out.at[c, pl.ds(s * ROWS_PER_TILE,
                                               ROWS_PER_TILE)])


@functools.cache
def _sc_embed_deg():
    return pl.kernel(
        _sc_embed_deg_body,
        out_type=[jax.ShapeDtypeStruct((NP, HID), _f32),
                  jax.ShapeDtypeStruct((2, NP), _f32)],
        mesh=_mesh(),
        scratch_types=[
            pltpu.VMEM((4, 80), jnp.int32),
            pltpu.VMEM((80, 128), jnp.int32),
            pltpu.VMEM((128, HID), _f32),
            pltpu.VMEM((128,), _f32),
            pltpu.VMEM((ROWS_PER_TILE,), _f32),
            pltpu.VMEM_SHARED((NP,), _f32),
            pltpu.SemaphoreType.DMA,
        ],
    )


_NBUF = 5


def _sc_prop_body(x_hbm, src_hbm, dst_hbm, out_hbm,
                  srcv, dstv, b0, b1, b2, b3, b4, acc,
                  g0, g1, g2, g3, g4, ssem):
    bufs = (b0, b1, b2, b3, b4)
    gsems = (g0, g1, g2, g3, g4)
    c = lax.axis_index("c")
    s = lax.axis_index("s")
    rowbase = s * ROWS_PER_TILE

    _zero_vmem_2d(bufs[0], 128, FH)
    for j in range(5):
        pltpu.sync_copy(bufs[0], acc.at[pl.ds(rowbase + j * 128, 128)])
    plsc.subcore_barrier()

    pltpu.sync_copy(src_hbm.at[s], srcv)
    pltpu.sync_copy(dst_hbm.at[s], dstv)

    xview = x_hbm.at[pl.ds(c * NP, NP)]
    drain_src = x_hbm.at[pl.ds(0, 128)]

    @pl.loop(0, 160, step=_NBUF)
    def _(jj):
        @pl.when(jj > 0)
        def _():
            for b in range(_NBUF):
                pltpu.make_async_copy(drain_src, bufs[b], ssem).wait()
        descs = [pltpu.async_copy(xview.at[srcv.at[jj + b]], bufs[b],
                                  gsems[b]) for b in range(_NBUF)]
        for b in range(_NBUF):
            descs[b].wait()
            pltpu.async_copy(bufs[b], acc.at[dstv.at[jj + b]], ssem,
                             add=True)

    for b in range(_NBUF):
        pltpu.make_async_copy(drain_src, bufs[b], ssem).wait()
    plsc.subcore_barrier()

    for j in range(5):
        pltpu.sync_copy(acc.at[pl.ds(rowbase + j * 128, 128)], bufs[0])
        pltpu.sync_copy(bufs[0], out_hbm.at[c, pl.ds(rowbase + j * 128, 128)])


@functools.cache
def _sc_prop():
    return pl.kernel(
        _sc_prop_body,
        out_type=jax.ShapeDtypeStruct((2, NP, FH), _f32),
        mesh=_mesh(),
        scratch_types=[
            pltpu.VMEM((160, 128), jnp.int32),
            pltpu.VMEM((160, 128), jnp.int32),
        ] + [pltpu.VMEM((128, FH), _f32)] * _NBUF + [
            pltpu.VMEM_SHARED((NP, FH), _f32),
        ] + [pltpu.SemaphoreType.DMA] * _NBUF + [
            pltpu.SemaphoreType.DMA,
        ],
        compiler_params=pltpu.CompilerParams(use_tc_tiling_on_sc=False),
    )


_BLK = 1024
_GRID = NP // _BLK


def _norm_from_deg(d0, d1):
    deg = jnp.maximum(d0 + d1, 1.0)
    return lax.rsqrt(deg)


def _tc1_body(g_ref, d0_ref, d1_ref, w_ref, b_ref, o_ref):
    norm = _norm_from_deg(d0_ref[...], d1_ref[...])
    x = jnp.dot(g_ref[...], w_ref[0], preferred_element_type=_f32)
    o_ref[...] = (x + b_ref[0]) * norm


def _tc2_body(p_ref, d0_ref, d1_ref, o_ref):
    norm = _norm_from_deg(d0_ref[...], d1_ref[...])
    o_ref[...] = p_ref[0] * (norm * norm)


def _tc3_body(q_ref, d0_ref, d1_ref, wp_ref, bp_ref,
              w0_ref, b0_ref, w1_ref, b1_ref, w2_ref, b2_ref, o_ref):
    norm = _norm_from_deg(d0_ref[...], d1_ref[...])
    x = jnp.concatenate([q_ref[0], q_ref[1]], axis=1) * norm
    z = jnp.dot(x, wp_ref[...], preferred_element_type=_f32) + bp_ref[...]
    a = jnp.maximum(jnp.dot(z, w0_ref[...], preferred_element_type=_f32)
                    + b0_ref[...], 0.0)
    b = jnp.maximum(jnp.dot(a, w1_ref[...], preferred_element_type=_f32)
                    + b1_ref[...], 0.0)
    o_ref[...] = jnp.dot(b, w2_ref[...], preferred_element_type=_f32) \
        + b2_ref[...]


def _row_spec(width):
    return pl.BlockSpec((_BLK, width), lambda i: (i, 0))


def _full_spec(shape):
    return pl.BlockSpec(shape, lambda i: tuple(0 for _ in shape))


_tc1 = pl.pallas_call(
    _tc1_body,
    grid=(2, _GRID),
    in_specs=[
        pl.BlockSpec((_BLK, HID), lambda c, g: (g, 0)),
        pl.BlockSpec((_BLK, 1), lambda c, g: (g, 0)),
        pl.BlockSpec((_BLK, 1), lambda c, g: (g, 0)),
        pl.BlockSpec((1, HID, FH), lambda c, g: (c, 0, 0)),
        pl.BlockSpec((1, 1, FH), lambda c, g: (c, 0, 0)),
    ],
    out_specs=pl.BlockSpec((_BLK, FH), lambda c, g: (c * _GRID + g, 0)),
    out_shape=jax.ShapeDtypeStruct((2 * NP, FH), _f32),
)

_tc2 = pl.pallas_call(
    _tc2_body,
    grid=(2, _GRID),
    in_specs=[
        pl.BlockSpec((1, _BLK, FH), lambda c, g: (c, g, 0)),
        pl.BlockSpec((_BLK, 1), lambda c, g: (g, 0)),
        pl.BlockSpec((_BLK, 1), lambda c, g: (g, 0)),
    ],
    out_specs=pl.BlockSpec((_BLK, FH), lambda c, g: (c * _GRID + g, 0)),
    out_shape=jax.ShapeDtypeStruct((2 * NP, FH), _f32),
)

_tc3 = pl.pallas_call(
    _tc3_body,
    grid=(_GRID,),
    in_specs=[pl.BlockSpec((2, _BLK, FH), lambda g: (0, g, 0)),
              pl.BlockSpec((_BLK, 1), lambda g: (g, 0)),
              pl.BlockSpec((_BLK, 1), lambda g: (g, 0))]
    + [_full_spec((HID, HID)), _full_spec((1, HID))] * 4,
    out_specs=_row_spec(HID),
    out_shape=jax.ShapeDtypeStruct((NP, HID), _f32),
)


def _pad_w(w):
    r, ccols = w.shape
    return jnp.pad(w, ((0, HID - r), (0, HID - ccols)))


def _pad_b(b):
    return jnp.pad(b, (0, HID - b.shape[0])).reshape(1, HID)


def kernel(h, edge_index, e, snorm_n, snorm_e, emb, W1, b1, Wp, bp,
           Wm0, bm0, Wm1, bm1, Wm2, bm2):
    h_p = jnp.pad(h.astype(jnp.int32), (0, NP - N)).reshape(NTILES, 4, 80)
    src = edge_index[0].astype(jnp.int32)
    dst = edge_index[1].astype(jnp.int32)
    src_p = jnp.pad(src, (0, EP - E))
    dst_p = jnp.pad(dst, (0, EP - E), constant_values=NP - 1)
    src16 = src_p.reshape(16, 160, 128)
    dst16 = dst_p.reshape(16, 160, 128)
    dst32 = dst_p.reshape(NTILES, 80, 128)

    g, degp = _sc_embed_deg()(emb, h_p, dst32)
    d0 = degp[0].reshape(NP, 1)
    d1 = degp[1].reshape(NP, 1)

    W1s = jnp.stack([W1[:, :FH], W1[:, FH:]])
    b1s = jnp.stack([b1[:FH], b1[FH:]]).reshape(2, 1, FH)
    xs = _tc1(g, d0, d1, W1s, b1s)

    p = _sc_prop()(xs, src16, dst16)
    xs2 = _tc2(p, d0, d1)
    q = _sc_prop()(xs2, src16, dst16)

    outp = _tc3(q, d0, d1, Wp, bp.reshape(1, HID),
                _pad_w(Wm0), _pad_b(bm0), _pad_w(Wm1), _pad_b(bm1),
                _pad_w(Wm2), _pad_b(bm2))
    return outp[:N, :NC_OUT]


# speedup vs baseline: 6.6515x; 1.5381x over previous
import functools

import jax
import jax.numpy as jnp
from jax import lax
from jax.experimental import pallas as pl
from jax.experimental.pallas import tpu as pltpu
from jax.experimental.pallas import tpu_sc as plsc

N = 10000
E = 320000
HID = 128
FH = HID // 2
NC_OUT = 6

NTILES = 32
NP = 10240
EP = 16 * 160 * 128
ROWS_PER_TILE = NP // 16

_f32 = jnp.float32


def _mesh():
    return plsc.VectorSubcoreMesh(core_axis_name="c", subcore_axis_name="s")


def _zero_vmem_2d(buf, nrows, width):
    @pl.loop(0, nrows)
    def _(i):
        for k in range(width // 16):
            buf[i, pl.ds(k * 16, 16)] = jnp.zeros((16,), _f32)


def _sc_embed_deg_body(emb_hbm, hp_hbm, dst_hbm, g_out, degp_out,
                       hidx_v, dstv, buf, ones_v, dbuf, acc1, sem):
    c = lax.axis_index("c")
    s = lax.axis_index("s")
    t = c * 16 + s

    @pl.loop(0, ROWS_PER_TILE // 16)
    def _(i):
        dbuf[pl.ds(i * 16, 16)] = jnp.zeros((16,), _f32)
    pltpu.sync_copy(dbuf, acc1.at[pl.ds(s * ROWS_PER_TILE, ROWS_PER_TILE)])
    for k in range(8):
        ones_v[pl.ds(k * 16, 16)] = jnp.ones((16,), _f32)
    plsc.subcore_barrier()

    pltpu.sync_copy(hp_hbm.at[t], hidx_v)
    @pl.loop(0, 4)
    def _(j):
        pltpu.async_copy(emb_hbm.at[hidx_v.at[j]], buf.at[pl.ds(0, 80)],
                         sem).wait()
        pltpu.sync_copy(buf.at[pl.ds(0, 80)],
                        g_out.at[pl.ds(t * 320 + j * 80, 80)])

    pltpu.sync_copy(dst_hbm.at[t], dstv)
    @pl.loop(0, 80)
    def _(j):
        pltpu.sync_copy(ones_v, acc1.at[dstv.at[j]], add=True)
    plsc.subcore_barrier()

    pltpu.sync_copy(acc1.at[pl.ds(s * ROWS_PER_TILE, ROWS_PER_TILE)], dbuf)
    pltpu.sync_copy(dbuf, degp_out.at[c, pl.ds(s * ROWS_PER_TILE,
                                               ROWS_PER_TILE)])


@functools.cache
def _sc_embed_deg():
    return pl.kernel(
        _sc_embed_deg_body,
        out_type=[jax.ShapeDtypeStruct((NP, HID), _f32),
                  jax.ShapeDtypeStruct((2, NP), _f32)],
        mesh=_mesh(),
        scratch_types=[
            pltpu.VMEM((4, 80), jnp.int32),
            pltpu.VMEM((80, 128), jnp.int32),
            pltpu.VMEM((128, HID), _f32),
            pltpu.VMEM((128,), _f32),
            pltpu.VMEM((ROWS_PER_TILE,), _f32),
            pltpu.VMEM_SHARED((NP,), _f32),
            pltpu.SemaphoreType.DMA,
        ],
    )


_NBUF = 5
FQ = HID // 4


def _sc_prop_body(x_hbm, src_hbm, dst_hbm, out_hbm,
                  srcv, dstv, b0, b1, b2, b3, b4, xq, acc,
                  g0, g1, g2, g3, g4, ssem):
    bufs = (b0, b1, b2, b3, b4)
    gsems = (g0, g1, g2, g3, g4)
    c = lax.axis_index("c")
    s = lax.axis_index("s")
    rowbase = s * ROWS_PER_TILE

    pltpu.sync_copy(src_hbm.at[s], srcv)
    pltpu.sync_copy(dst_hbm.at[s], dstv)

    drain_src = x_hbm.at[pl.ds(0, 128)]

    for sub in range(2):
        q = c * 2 + sub

        pltpu.sync_copy(x_hbm.at[pl.ds(q * NP + rowbase, ROWS_PER_TILE)],
                        xq.at[pl.ds(rowbase, ROWS_PER_TILE)])
        _zero_vmem_2d(bufs[0], 128, FQ)
        for j in range(5):
            pltpu.sync_copy(bufs[0], acc.at[pl.ds(rowbase + j * 128, 128)])
        plsc.subcore_barrier()

        @pl.loop(0, 160, step=_NBUF)
        def _(jj):
            @pl.when(jj > 0)
            def _():
                for b in range(_NBUF):
                    pltpu.make_async_copy(drain_src, bufs[b], ssem).wait()
            descs = [pltpu.async_copy(xq.at[srcv.at[jj + b]], bufs[b],
                                      gsems[b]) for b in range(_NBUF)]
            for b in range(_NBUF):
                descs[b].wait()
                pltpu.async_copy(bufs[b], acc.at[dstv.at[jj + b]], ssem,
                                 add=True)

        for b in range(_NBUF):
            pltpu.make_async_copy(drain_src, bufs[b], ssem).wait()
        plsc.subcore_barrier()

        for j in range(5):
            pltpu.sync_copy(acc.at[pl.ds(rowbase + j * 128, 128)], bufs[0])
            pltpu.sync_copy(bufs[0],
                            out_hbm.at[q, pl.ds(rowbase + j * 128, 128)])


@functools.cache
def _sc_prop():
    return pl.kernel(
        _sc_prop_body,
        out_type=jax.ShapeDtypeStruct((4, NP, FQ), _f32),
        mesh=_mesh(),
        scratch_types=[
            pltpu.VMEM((160, 128), jnp.int32),
            pltpu.VMEM((160, 128), jnp.int32),
        ] + [pltpu.VMEM((128, FQ), _f32)] * _NBUF + [
            pltpu.VMEM_SHARED((NP, FQ), _f32),
            pltpu.VMEM_SHARED((NP, FQ), _f32),
        ] + [pltpu.SemaphoreType.DMA] * _NBUF + [
            pltpu.SemaphoreType.DMA,
        ],
        compiler_params=pltpu.CompilerParams(use_tc_tiling_on_sc=False),
    )


_BLK = 1024
_GRID = NP // _BLK


def _norm_from_deg(d0, d1):
    deg = jnp.maximum(d0 + d1, 1.0)
    return lax.rsqrt(deg)


def _tc1_body(g_ref, d0_ref, d1_ref, w_ref, b_ref, o_ref):
    norm = _norm_from_deg(d0_ref[...], d1_ref[...])
    x = jnp.dot(g_ref[...], w_ref[0], preferred_element_type=_f32)
    o_ref[...] = (x + b_ref[0]) * norm


def _tc2_body(p_ref, d0_ref, d1_ref, o_ref):
    norm = _norm_from_deg(d0_ref[...], d1_ref[...])
    o_ref[...] = p_ref[0] * (norm * norm)


def _tc3_body(q_ref, d0_ref, d1_ref, wp_ref, bp_ref,
              w0_ref, b0_ref, w1_ref, b1_ref, w2_ref, b2_ref, o_ref):
    norm = _norm_from_deg(d0_ref[...], d1_ref[...])
    x = jnp.concatenate([q_ref[i] for i in range(4)], axis=1) * norm
    z = jnp.dot(x, wp_ref[...], preferred_element_type=_f32) + bp_ref[...]
    a = jnp.maximum(jnp.dot(z, w0_ref[...], preferred_element_type=_f32)
                    + b0_ref[...], 0.0)
    b = jnp.maximum(jnp.dot(a, w1_ref[...], preferred_element_type=_f32)
                    + b1_ref[...], 0.0)
    o_ref[...] = jnp.dot(b, w2_ref[...], preferred_element_type=_f32) \
        + b2_ref[...]


def _row_spec(width):
    return pl.BlockSpec((_BLK, width), lambda i: (i, 0))


def _full_spec(shape):
    return pl.BlockSpec(shape, lambda i: tuple(0 for _ in shape))


_tc1 = pl.pallas_call(
    _tc1_body,
    grid=(4, _GRID),
    in_specs=[
        pl.BlockSpec((_BLK, HID), lambda c, g: (g, 0)),
        pl.BlockSpec((_BLK, 1), lambda c, g: (g, 0)),
        pl.BlockSpec((_BLK, 1), lambda c, g: (g, 0)),
        pl.BlockSpec((1, HID, FQ), lambda c, g: (c, 0, 0)),
        pl.BlockSpec((1, 1, FQ), lambda c, g: (c, 0, 0)),
    ],
    out_specs=pl.BlockSpec((_BLK, FQ), lambda c, g: (c * _GRID + g, 0)),
    out_shape=jax.ShapeDtypeStruct((4 * NP, FQ), _f32),
)

_tc2 = pl.pallas_call(
    _tc2_body,
    grid=(4, _GRID),
    in_specs=[
        pl.BlockSpec((1, _BLK, FQ), lambda c, g: (c, g, 0)),
        pl.BlockSpec((_BLK, 1), lambda c, g: (g, 0)),
        pl.BlockSpec((_BLK, 1), lambda c, g: (g, 0)),
    ],
    out_specs=pl.BlockSpec((_BLK, FQ), lambda c, g: (c * _GRID + g, 0)),
    out_shape=jax.ShapeDtypeStruct((4 * NP, FQ), _f32),
)

_tc3 = pl.pallas_call(
    _tc3_body,
    grid=(_GRID,),
    in_specs=[pl.BlockSpec((4, _BLK, FQ), lambda g: (0, g, 0)),
              pl.BlockSpec((_BLK, 1), lambda g: (g, 0)),
              pl.BlockSpec((_BLK, 1), lambda g: (g, 0))]
    + [_full_spec((HID, HID)), _full_spec((1, HID))] * 4,
    out_specs=_row_spec(HID),
    out_shape=jax.ShapeDtypeStruct((NP, HID), _f32),
)


def _pad_w(w):
    r, ccols = w.shape
    return jnp.pad(w, ((0, HID - r), (0, HID - ccols)))


def _pad_b(b):
    return jnp.pad(b, (0, HID - b.shape[0])).reshape(1, HID)


def kernel(h, edge_index, e, snorm_n, snorm_e, emb, W1, b1, Wp, bp,
           Wm0, bm0, Wm1, bm1, Wm2, bm2):
    h_p = jnp.pad(h.astype(jnp.int32), (0, NP - N)).reshape(NTILES, 4, 80)
    src = edge_index[0].astype(jnp.int32)
    dst = edge_index[1].astype(jnp.int32)
    src_p = jnp.pad(src, (0, EP - E))
    dst_p = jnp.pad(dst, (0, EP - E), constant_values=NP - 1)
    src16 = src_p.reshape(16, 160, 128)
    dst16 = dst_p.reshape(16, 160, 128)
    dst32 = dst_p.reshape(NTILES, 80, 128)

    g, degp = _sc_embed_deg()(emb, h_p, dst32)
    d0 = degp[0].reshape(NP, 1)
    d1 = degp[1].reshape(NP, 1)

    W1s = jnp.stack([W1[:, i * FQ:(i + 1) * FQ] for i in range(4)])
    b1s = jnp.stack([b1[i * FQ:(i + 1) * FQ]
                     for i in range(4)]).reshape(4, 1, FQ)
    xs = _tc1(g, d0, d1, W1s, b1s)

    p = _sc_prop()(xs, src16, dst16)
    xs2 = _tc2(p, d0, d1)
    q = _sc_prop()(xs2, src16, dst16)

    outp = _tc3(q, d0, d1, Wp, bp.reshape(1, HID),
                _pad_w(Wm0), _pad_b(bm0), _pad_w(Wm1), _pad_b(bm1),
                _pad_w(Wm2), _pad_b(bm2))
    return outp[:N, :NC_OUT]


# speedup vs baseline: 7.4804x; 1.1246x over previous
import functools

import jax
import jax.numpy as jnp
from jax import lax
from jax.experimental import pallas as pl
from jax.experimental.pallas import tpu as pltpu
from jax.experimental.pallas import tpu_sc as plsc

N = 10000
E = 320000
HID = 128
FH = HID // 2
NC_OUT = 6

NTILES = 32
NP = 10240
EP = 16 * 160 * 128
ROWS_PER_TILE = NP // 16

_f32 = jnp.float32


def _mesh():
    return plsc.VectorSubcoreMesh(core_axis_name="c", subcore_axis_name="s")


def _zero_vmem_2d(buf, nrows, width):
    @pl.loop(0, nrows)
    def _(i):
        for k in range(width // 16):
            buf[i, pl.ds(k * 16, 16)] = jnp.zeros((16,), _f32)


def _sc_embed_deg_body(emb_hbm, hp_hbm, dst_hbm, g_out, degp_out,
                       hidx_v, dstv, buf, ones_v, dbuf, acc1, sem):
    c = lax.axis_index("c")
    s = lax.axis_index("s")
    t = c * 16 + s

    @pl.loop(0, ROWS_PER_TILE // 16)
    def _(i):
        dbuf[pl.ds(i * 16, 16)] = jnp.zeros((16,), _f32)
    pltpu.sync_copy(dbuf, acc1.at[pl.ds(s * ROWS_PER_TILE, ROWS_PER_TILE)])
    for k in range(8):
        ones_v[pl.ds(k * 16, 16)] = jnp.ones((16,), _f32)
    plsc.subcore_barrier()

    pltpu.sync_copy(hp_hbm.at[t], hidx_v)
    @pl.loop(0, 4)
    def _(j):
        pltpu.async_copy(emb_hbm.at[hidx_v.at[j]], buf.at[pl.ds(0, 80)],
                         sem).wait()
        pltpu.sync_copy(buf.at[pl.ds(0, 80)],
                        g_out.at[pl.ds(t * 320 + j * 80, 80)])

    pltpu.sync_copy(dst_hbm.at[t], dstv)
    @pl.loop(0, 80)
    def _(j):
        pltpu.sync_copy(ones_v, acc1.at[dstv.at[j]], add=True)
    plsc.subcore_barrier()

    pltpu.sync_copy(acc1.at[pl.ds(s * ROWS_PER_TILE, ROWS_PER_TILE)], dbuf)
    pltpu.sync_copy(dbuf, degp_out.at[c, pl.ds(s * ROWS_PER_TILE,
                                               ROWS_PER_TILE)])


@functools.cache
def _sc_embed_deg():
    return pl.kernel(
        _sc_embed_deg_body,
        out_type=[jax.ShapeDtypeStruct((NP, HID), _f32),
                  jax.ShapeDtypeStruct((2, NP), _f32)],
        mesh=_mesh(),
        scratch_types=[
            pltpu.VMEM((4, 80), jnp.int32),
            pltpu.VMEM((80, 128), jnp.int32),
            pltpu.VMEM((128, HID), _f32),
            pltpu.VMEM((128,), _f32),
            pltpu.VMEM((ROWS_PER_TILE,), _f32),
            pltpu.VMEM_SHARED((NP,), _f32),
            pltpu.SemaphoreType.DMA,
        ],
    )


_NBUF = 5
FQ = HID // 4


def _sc_prop2_body(x_hbm, n2_hbm, src_hbm, dst_hbm, out_hbm,
                   srcv, dstv, n2v, b0, b1, b2, b3, b4, xq, accA, accB,
                   g0, g1, g2, g3, g4, ssem):
    bufs = (b0, b1, b2, b3, b4)
    gsems = (g0, g1, g2, g3, g4)
    c = lax.axis_index("c")
    s = lax.axis_index("s")
    rowbase = s * ROWS_PER_TILE
    qA = c * 2
    qB = c * 2 + 1

    pltpu.sync_copy(src_hbm.at[s], srcv)
    pltpu.sync_copy(dst_hbm.at[s], dstv)
    pltpu.sync_copy(n2_hbm.at[pl.ds(rowbase, ROWS_PER_TILE)],
                    n2v.at[pl.ds(0, ROWS_PER_TILE)])

    drain_src = x_hbm.at[pl.ds(0, 128)]

    def zero_slab(slab):
        _zero_vmem_2d(bufs[0], 128, FQ)
        for j in range(5):
            pltpu.sync_copy(bufs[0], slab.at[pl.ds(rowbase + j * 128, 128)])

    def gather_scatter(src_slab, acc_slab):
        @pl.loop(0, 160, step=_NBUF)
        def _(jj):
            @pl.when(jj > 0)
            def _():
                for b in range(_NBUF):
                    pltpu.make_async_copy(drain_src, bufs[b], ssem).wait()
            descs = [pltpu.async_copy(src_slab.at[srcv.at[jj + b]], bufs[b],
                                      gsems[b]) for b in range(_NBUF)]
            for b in range(_NBUF):
                descs[b].wait()
                pltpu.async_copy(bufs[b], acc_slab.at[dstv.at[jj + b]], ssem,
                                 add=True)
        for b in range(_NBUF):
            pltpu.make_async_copy(drain_src, bufs[b], ssem).wait()

    def copy_out(slab, q):
        for j in range(5):
            pltpu.sync_copy(slab.at[pl.ds(rowbase + j * 128, 128)], bufs[0])
            pltpu.sync_copy(bufs[0],
                            out_hbm.at[q, pl.ds(rowbase + j * 128, 128)])

    pltpu.sync_copy(x_hbm.at[pl.ds(qA * NP + rowbase, ROWS_PER_TILE)],
                    xq.at[pl.ds(rowbase, ROWS_PER_TILE)])
    zero_slab(accA)
    plsc.subcore_barrier()
    gather_scatter(xq, accA)
    plsc.subcore_barrier()

    pltpu.sync_copy(x_hbm.at[pl.ds(qB * NP + rowbase, ROWS_PER_TILE)],
                    xq.at[pl.ds(rowbase, ROWS_PER_TILE)])
    zero_slab(accB)
    plsc.subcore_barrier()
    gather_scatter(xq, accB)
    plsc.subcore_barrier()

    for j in range(5):
        pltpu.sync_copy(accA.at[pl.ds(rowbase + j * 128, 128)], bufs[1])
        pltpu.sync_copy(accB.at[pl.ds(rowbase + j * 128, 128)], bufs[2])

        @pl.loop(0, 128)
        def _(i):
            n = n2v[pl.ds(j * 128 + i, 16)][0]
            for k in range(FQ // 16):
                bufs[1][i, pl.ds(k * 16, 16)] = \
                    bufs[1][i, pl.ds(k * 16, 16)] * n
                bufs[2][i, pl.ds(k * 16, 16)] = \
                    bufs[2][i, pl.ds(k * 16, 16)] * n
        pltpu.sync_copy(bufs[1], accA.at[pl.ds(rowbase + j * 128, 128)])
        pltpu.sync_copy(bufs[2], accB.at[pl.ds(rowbase + j * 128, 128)])
    zero_slab(xq)
    plsc.subcore_barrier()

    gather_scatter(accA, xq)
    plsc.subcore_barrier()
    copy_out(xq, qA)
    zero_slab(accA)
    plsc.subcore_barrier()

    gather_scatter(accB, accA)
    plsc.subcore_barrier()
    copy_out(accA, qB)


@functools.cache
def _sc_prop2():
    return pl.kernel(
        _sc_prop2_body,
        out_type=jax.ShapeDtypeStruct((4, NP, FQ), _f32),
        mesh=_mesh(),
        scratch_types=[
            pltpu.VMEM((160, 128), jnp.int32),
            pltpu.VMEM((160, 128), jnp.int32),
            pltpu.VMEM((ROWS_PER_TILE + 16,), _f32),
        ] + [pltpu.VMEM((128, FQ), _f32)] * _NBUF + [
            pltpu.VMEM_SHARED((NP, FQ), _f32),
            pltpu.VMEM_SHARED((NP, FQ), _f32),
            pltpu.VMEM_SHARED((NP, FQ), _f32),
        ] + [pltpu.SemaphoreType.DMA] * _NBUF + [
            pltpu.SemaphoreType.DMA,
        ],
        compiler_params=pltpu.CompilerParams(use_tc_tiling_on_sc=False),
    )


_BLK = 1024
_GRID = NP // _BLK


def _norm_from_deg(d0, d1):
    deg = jnp.maximum(d0 + d1, 1.0)
    return lax.rsqrt(deg)


def _tc1_body(g_ref, d0_ref, d1_ref, w_ref, b_ref, o_ref, n2_ref):
    norm = _norm_from_deg(d0_ref[...], d1_ref[...])
    x = jnp.dot(g_ref[...], w_ref[0], preferred_element_type=_f32)
    o_ref[...] = (x + b_ref[0]) * norm
    n2_ref[...] = norm * norm


def _tc3_body(q_ref, d0_ref, d1_ref, wp_ref, bp_ref,
              w0_ref, b0_ref, w1_ref, b1_ref, w2_ref, b2_ref, o_ref):
    norm = _norm_from_deg(d0_ref[...], d1_ref[...])
    x = jnp.concatenate([q_ref[i] for i in range(4)], axis=1) * norm
    z = jnp.dot(x, wp_ref[...], preferred_element_type=_f32) + bp_ref[...]
    a = jnp.maximum(jnp.dot(z, w0_ref[...], preferred_element_type=_f32)
                    + b0_ref[...], 0.0)
    b = jnp.maximum(jnp.dot(a, w1_ref[...], preferred_element_type=_f32)
                    + b1_ref[...], 0.0)
    o_ref[...] = jnp.dot(b, w2_ref[...], preferred_element_type=_f32) \
        + b2_ref[...]


def _row_spec(width):
    return pl.BlockSpec((_BLK, width), lambda i: (i, 0))


def _full_spec(shape):
    return pl.BlockSpec(shape, lambda i: tuple(0 for _ in shape))


_tc1 = pl.pallas_call(
    _tc1_body,
    grid=(4, _GRID),
    in_specs=[
        pl.BlockSpec((_BLK, HID), lambda c, g: (g, 0)),
        pl.BlockSpec((_BLK, 1), lambda c, g: (g, 0)),
        pl.BlockSpec((_BLK, 1), lambda c, g: (g, 0)),
        pl.BlockSpec((1, HID, FQ), lambda c, g: (c, 0, 0)),
        pl.BlockSpec((1, 1, FQ), lambda c, g: (c, 0, 0)),
    ],
    out_specs=[pl.BlockSpec((_BLK, FQ), lambda c, g: (c * _GRID + g, 0)),
               pl.BlockSpec((_BLK, 1), lambda c, g: (g, 0))],
    out_shape=[jax.ShapeDtypeStruct((4 * NP, FQ), _f32),
               jax.ShapeDtypeStruct((NP, 1), _f32)],
)

_tc3 = pl.pallas_call(
    _tc3_body,
    grid=(_GRID,),
    in_specs=[pl.BlockSpec((4, _BLK, FQ), lambda g: (0, g, 0)),
              pl.BlockSpec((_BLK, 1), lambda g: (g, 0)),
              pl.BlockSpec((_BLK, 1), lambda g: (g, 0))]
    + [_full_spec((HID, HID)), _full_spec((1, HID))] * 4,
    out_specs=_row_spec(HID),
    out_shape=jax.ShapeDtypeStruct((NP, HID), _f32),
)


def _pad_w(w):
    r, ccols = w.shape
    return jnp.pad(w, ((0, HID - r), (0, HID - ccols)))


def _pad_b(b):
    return jnp.pad(b, (0, HID - b.shape[0])).reshape(1, HID)


def kernel(h, edge_index, e, snorm_n, snorm_e, emb, W1, b1, Wp, bp,
           Wm0, bm0, Wm1, bm1, Wm2, bm2):
    h_p = jnp.pad(h.astype(jnp.int32), (0, NP - N)).reshape(NTILES, 4, 80)
    src = edge_index[0].astype(jnp.int32)
    dst = edge_index[1].astype(jnp.int32)
    src_p = jnp.pad(src, (0, EP - E))
    dst_p = jnp.pad(dst, (0, EP - E), constant_values=NP - 1)
    src16 = src_p.reshape(16, 160, 128)
    dst16 = dst_p.reshape(16, 160, 128)
    dst32 = dst_p.reshape(NTILES, 80, 128)

    g, degp = _sc_embed_deg()(emb, h_p, dst32)
    d0 = degp[0].reshape(NP, 1)
    d1 = degp[1].reshape(NP, 1)

    W1s = jnp.stack([W1[:, i * FQ:(i + 1) * FQ] for i in range(4)])
    b1s = jnp.stack([b1[i * FQ:(i + 1) * FQ]
                     for i in range(4)]).reshape(4, 1, FQ)
    xs, n2 = _tc1(g, d0, d1, W1s, b1s)

    q = _sc_prop2()(xs, n2.reshape(NP), src16, dst16)

    outp = _tc3(q, d0, d1, Wp, bp.reshape(1, HID),
                _pad_w(Wm0), _pad_b(bm0), _pad_w(Wm1), _pad_b(bm1),
                _pad_w(Wm2), _pad_b(bm2))
    return outp[:N, :NC_OUT]


# speedup vs baseline: 7.6209x; 1.0188x over previous
import functools

import jax
import jax.numpy as jnp
from jax import lax
from jax.experimental import pallas as pl
from jax.experimental.pallas import tpu as pltpu
from jax.experimental.pallas import tpu_sc as plsc

N = 10000
E = 320000
HID = 128
FH = HID // 2
NC_OUT = 6

NTILES = 32
NP = 10240
EP = 16 * 160 * 128
ROWS_PER_TILE = NP // 16

_f32 = jnp.float32


def _mesh():
    return plsc.VectorSubcoreMesh(core_axis_name="c", subcore_axis_name="s")


def _zero_vmem_2d(buf, nrows, width):
    @pl.loop(0, nrows)
    def _(i):
        for k in range(width // 16):
            buf[i, pl.ds(k * 16, 16)] = jnp.zeros((16,), _f32)


def _sc_embed_deg_body(emb_hbm, hp_hbm, dst_hbm, g_out, degp_out,
                       hidx_v, dstv, buf, ones_v, dbuf, acc1,
                       e0, e1, e2, e3, ssem):
    esems = (e0, e1, e2, e3)
    c = lax.axis_index("c")
    s = lax.axis_index("s")
    t = c * 16 + s

    @pl.loop(0, ROWS_PER_TILE // 16)
    def _(i):
        dbuf[pl.ds(i * 16, 16)] = jnp.zeros((16,), _f32)
    pltpu.sync_copy(dbuf, acc1.at[pl.ds(s * ROWS_PER_TILE, ROWS_PER_TILE)])
    for k in range(8):
        ones_v[pl.ds(k * 16, 16)] = jnp.ones((16,), _f32)
    plsc.subcore_barrier()

    pltpu.sync_copy(hp_hbm.at[t], hidx_v)
    descs = [pltpu.async_copy(emb_hbm.at[hidx_v.at[j]],
                              buf.at[pl.ds(j * 80, 80)], esems[j])
             for j in range(4)]

    pltpu.sync_copy(dst_hbm.at[t], dstv)
    @pl.loop(0, 80)
    def _(j):
        pltpu.async_copy(ones_v, acc1.at[dstv.at[j]], ssem, add=True)

    for j in range(4):
        descs[j].wait()
        pltpu.sync_copy(buf.at[pl.ds(j * 80, 80)],
                        g_out.at[pl.ds(t * 320 + j * 80, 80)])

    @pl.loop(0, 80)
    def _(j):
        pltpu.make_async_copy(ones_v, acc1.at[dstv.at[0]], ssem).wait()
    plsc.subcore_barrier()

    pltpu.sync_copy(acc1.at[pl.ds(s * ROWS_PER_TILE, ROWS_PER_TILE)], dbuf)
    pltpu.sync_copy(dbuf, degp_out.at[c, pl.ds(s * ROWS_PER_TILE,
                                               ROWS_PER_TILE)])


@functools.cache
def _sc_embed_deg():
    return pl.kernel(
        _sc_embed_deg_body,
        out_type=[jax.ShapeDtypeStruct((NP, HID), _f32),
                  jax.ShapeDtypeStruct((2, NP), _f32)],
        mesh=_mesh(),
        scratch_types=[
            pltpu.VMEM((4, 80), jnp.int32),
            pltpu.VMEM((80, 128), jnp.int32),
            pltpu.VMEM((320, HID), _f32),
            pltpu.VMEM((128,), _f32),
            pltpu.VMEM((ROWS_PER_TILE,), _f32),
            pltpu.VMEM_SHARED((NP,), _f32),
        ] + [pltpu.SemaphoreType.DMA] * 5,
    )


_NBUF = 5
FQ = HID // 4


def _sc_prop2_body(x_hbm, n2_hbm, src_hbm, dst_hbm, out_hbm,
                   srcv, dstv, n2v, b0, b1, b2, b3, b4,
                   xq, accA, accB,
                   g0, g1, g2, g3, g4, ssem):
    bufs = (b0, b1, b2, b3, b4)
    gsems = (g0, g1, g2, g3, g4)
    c = lax.axis_index("c")
    s = lax.axis_index("s")
    rowbase = s * ROWS_PER_TILE
    qA = c * 2
    qB = c * 2 + 1

    pltpu.sync_copy(src_hbm.at[s], srcv)
    pltpu.sync_copy(dst_hbm.at[s], dstv)
    pltpu.sync_copy(n2_hbm.at[pl.ds(rowbase, ROWS_PER_TILE)],
                    n2v.at[pl.ds(0, ROWS_PER_TILE)])

    drain_src = x_hbm.at[pl.ds(0, 128)]

    def zero_slab(slab):
        _zero_vmem_2d(bufs[0], 128, FQ)
        for j in range(5):
            pltpu.sync_copy(bufs[0], slab.at[pl.ds(rowbase + j * 128, 128)])

    def gather_scatter(src_slab, acc_slab):
        @pl.loop(0, 160, step=_NBUF)
        def _(jj):
            @pl.when(jj > 0)
            def _():
                for b in range(_NBUF):
                    pltpu.make_async_copy(drain_src, bufs[b], ssem).wait()
            descs = [pltpu.async_copy(src_slab.at[srcv.at[jj + b]], bufs[b],
                                      gsems[b]) for b in range(_NBUF)]
            for b in range(_NBUF):
                descs[b].wait()
                pltpu.async_copy(bufs[b], acc_slab.at[dstv.at[jj + b]], ssem,
                                 add=True)
        for b in range(_NBUF):
            pltpu.make_async_copy(drain_src, bufs[b], ssem).wait()

    def copy_out(slab, q):
        for j in range(5):
            pltpu.sync_copy(slab.at[pl.ds(rowbase + j * 128, 128)], bufs[0])
            pltpu.sync_copy(bufs[0],
                            out_hbm.at[q, pl.ds(rowbase + j * 128, 128)])

    pltpu.sync_copy(x_hbm.at[pl.ds(qA * NP + rowbase, ROWS_PER_TILE)],
                    xq.at[pl.ds(rowbase, ROWS_PER_TILE)])
    zero_slab(accA)
    plsc.subcore_barrier()
    gather_scatter(xq, accA)
    plsc.subcore_barrier()

    pltpu.sync_copy(x_hbm.at[pl.ds(qB * NP + rowbase, ROWS_PER_TILE)],
                    xq.at[pl.ds(rowbase, ROWS_PER_TILE)])
    zero_slab(accB)
    plsc.subcore_barrier()
    gather_scatter(xq, accB)
    plsc.subcore_barrier()

    for j in range(5):
        pltpu.sync_copy(accA.at[pl.ds(rowbase + j * 128, 128)], bufs[1])
        pltpu.sync_copy(accB.at[pl.ds(rowbase + j * 128, 128)], bufs[2])

        @pl.loop(0, 128)
        def _(i):
            n = n2v[pl.ds(j * 128 + i, 16)][0]
            for k in range(FQ // 16):
                bufs[1][i, pl.ds(k * 16, 16)] = \
                    bufs[1][i, pl.ds(k * 16, 16)] * n
                bufs[2][i, pl.ds(k * 16, 16)] = \
                    bufs[2][i, pl.ds(k * 16, 16)] * n
        pltpu.sync_copy(bufs[1], accA.at[pl.ds(rowbase + j * 128, 128)])
        pltpu.sync_copy(bufs[2], accB.at[pl.ds(rowbase + j * 128, 128)])
    zero_slab(xq)
    plsc.subcore_barrier()

    gather_scatter(accA, xq)
    plsc.subcore_barrier()
    copy_out(xq, qA)
    zero_slab(accA)
    plsc.subcore_barrier()

    gather_scatter(accB, accA)
    plsc.subcore_barrier()
    copy_out(accA, qB)


@functools.cache
def _sc_prop2():
    return pl.kernel(
        _sc_prop2_body,
        out_type=jax.ShapeDtypeStruct((4, NP, FQ), _f32),
        mesh=_mesh(),
        scratch_types=[
            pltpu.VMEM((160, 128), jnp.int32),
            pltpu.VMEM((160, 128), jnp.int32),
            pltpu.VMEM((ROWS_PER_TILE + 16,), _f32),
        ] + [pltpu.VMEM((128, FQ), _f32)] * _NBUF + [
            pltpu.VMEM_SHARED((NP, FQ), _f32),
            pltpu.VMEM_SHARED((NP, FQ), _f32),
            pltpu.VMEM_SHARED((NP, FQ), _f32),
        ] + [pltpu.SemaphoreType.DMA] * _NBUF + [
            pltpu.SemaphoreType.DMA,
        ],
        compiler_params=pltpu.CompilerParams(use_tc_tiling_on_sc=False),
    )


_BLK = 1024
_GRID = NP // _BLK


def _norm_from_deg(d0, d1):
    deg = jnp.maximum(d0 + d1, 1.0)
    return lax.rsqrt(deg)


def _tc1_body(g_ref, d0_ref, d1_ref, w_ref, b_ref, o_ref, n2_ref):
    norm = _norm_from_deg(d0_ref[...], d1_ref[...])
    x = jnp.dot(g_ref[...], w_ref[0], preferred_element_type=_f32)
    o_ref[...] = (x + b_ref[0]) * norm
    n2_ref[...] = norm * norm


def _tc3_body(q_ref, d0_ref, d1_ref, wp_ref, bp_ref,
              w0_ref, b0_ref, w1_ref, b1_ref, w2_ref, b2_ref, o_ref):
    norm = _norm_from_deg(d0_ref[...], d1_ref[...])
    x = jnp.concatenate([q_ref[i] for i in range(4)], axis=1) * norm
    z = jnp.dot(x, wp_ref[...], preferred_element_type=_f32) + bp_ref[...]
    a = jnp.maximum(jnp.dot(z, w0_ref[...], preferred_element_type=_f32)
                    + b0_ref[...], 0.0)
    b = jnp.maximum(jnp.dot(a, w1_ref[...], preferred_element_type=_f32)
                    + b1_ref[...], 0.0)
    o_ref[...] = jnp.dot(b, w2_ref[...], preferred_element_type=_f32) \
        + b2_ref[...]


def _row_spec(width):
    return pl.BlockSpec((_BLK, width), lambda i: (i, 0))


def _full_spec(shape):
    return pl.BlockSpec(shape, lambda i: tuple(0 for _ in shape))


_tc1 = pl.pallas_call(
    _tc1_body,
    grid=(4, _GRID),
    in_specs=[
        pl.BlockSpec((_BLK, HID), lambda c, g: (g, 0)),
        pl.BlockSpec((_BLK, 1), lambda c, g: (g, 0)),
        pl.BlockSpec((_BLK, 1), lambda c, g: (g, 0)),
        pl.BlockSpec((1, HID, FQ), lambda c, g: (c, 0, 0)),
        pl.BlockSpec((1, 1, FQ), lambda c, g: (c, 0, 0)),
    ],
    out_specs=[pl.BlockSpec((_BLK, FQ), lambda c, g: (c * _GRID + g, 0)),
               pl.BlockSpec((_BLK, 1), lambda c, g: (g, 0))],
    out_shape=[jax.ShapeDtypeStruct((4 * NP, FQ), _f32),
               jax.ShapeDtypeStruct((NP, 1), _f32)],
)

_tc3 = pl.pallas_call(
    _tc3_body,
    grid=(_GRID,),
    in_specs=[pl.BlockSpec((4, _BLK, FQ), lambda g: (0, g, 0)),
              pl.BlockSpec((_BLK, 1), lambda g: (g, 0)),
              pl.BlockSpec((_BLK, 1), lambda g: (g, 0))]
    + [_full_spec((HID, HID)), _full_spec((1, HID))] * 4,
    out_specs=_row_spec(HID),
    out_shape=jax.ShapeDtypeStruct((NP, HID), _f32),
)


def _pad_w(w):
    r, ccols = w.shape
    return jnp.pad(w, ((0, HID - r), (0, HID - ccols)))


def _pad_b(b):
    return jnp.pad(b, (0, HID - b.shape[0])).reshape(1, HID)


def kernel(h, edge_index, e, snorm_n, snorm_e, emb, W1, b1, Wp, bp,
           Wm0, bm0, Wm1, bm1, Wm2, bm2):
    h_p = jnp.pad(h.astype(jnp.int32), (0, NP - N)).reshape(NTILES, 4, 80)
    src = edge_index[0].astype(jnp.int32)
    dst = edge_index[1].astype(jnp.int32)
    src_p = jnp.pad(src, (0, EP - E))
    dst_p = jnp.pad(dst, (0, EP - E), constant_values=NP - 1)
    src16 = src_p.reshape(16, 160, 128)
    dst16 = dst_p.reshape(16, 160, 128)
    dst32 = dst_p.reshape(NTILES, 80, 128)

    g, degp = _sc_embed_deg()(emb, h_p, dst32)
    d0 = degp[0].reshape(NP, 1)
    d1 = degp[1].reshape(NP, 1)

    W1s = jnp.stack([W1[:, i * FQ:(i + 1) * FQ] for i in range(4)])
    b1s = jnp.stack([b1[i * FQ:(i + 1) * FQ]
                     for i in range(4)]).reshape(4, 1, FQ)
    xs, n2 = _tc1(g, d0, d1, W1s, b1s)

    q = _sc_prop2()(xs, n2.reshape(NP), src16, dst16)

    outp = _tc3(q, d0, d1, Wp, bp.reshape(1, HID),
                _pad_w(Wm0), _pad_b(bm0), _pad_w(Wm1), _pad_b(bm1),
                _pad_w(Wm2), _pad_b(bm2))
    return outp[:N, :NC_OUT]
